# Initial kernel scaffold; baseline (speedup 1.0000x reference)
import jax
import jax.numpy as jnp
from jax.experimental import pallas as pl


def kernel(edge_index, edge_weight, y, inputs, sample_mask):
    raise NotImplementedError("write your pallas kernel here")


# speedup vs baseline: 63.8693x; 63.8693x over previous
import functools

import jax
import jax.numpy as jnp
from jax import lax
from jax.experimental import pallas as pl
from jax.experimental.pallas import tpu as pltpu
from jax.experimental.pallas import tpu_sc as plsc

_NC = 2
_NS = 16
_L = 16
_NW = _NC * _NS
_KB = 2000


def _sc_mesh():
    return plsc.VectorSubcoreMesh(
        core_axis_name="c", subcore_axis_name="s", num_cores=_NC, num_subcores=_NS
    )


def _sc_degree(row, w, npad, epw):
    zch = npad // _NS

    @functools.partial(
        pl.kernel,
        out_type=jax.ShapeDtypeStruct((_NC, npad), jnp.float32),
        mesh=_sc_mesh(),
        scratch_types=[
            pltpu.VMEM((epw,), jnp.int32),
            pltpu.VMEM((epw,), jnp.float32),
            pltpu.VMEM_SHARED((npad,), jnp.float32),
        ],
    )
    def deg_kernel(row_h, w_h, z_h, out_h, row_v, w_v, deg_sh):
        cid = lax.axis_index("c")
        sid = lax.axis_index("s")
        wid = cid * _NS + sid
        base = wid * epw
        pltpu.sync_copy(z_h, deg_sh.at[pl.ds(sid * zch, zch)])
        plsc.subcore_barrier()
        pltpu.sync_copy(row_h.at[pl.ds(base, epw)], row_v)
        pltpu.sync_copy(w_h.at[pl.ds(base, epw)], w_v)
        pltpu.sync_copy(w_v, deg_sh.at[row_v], add=True)
        plsc.subcore_barrier()
        pltpu.sync_copy(deg_sh.at[pl.ds(sid * zch, zch)], out_h.at[cid, pl.ds(sid * zch, zch)])

    zeros = jnp.zeros((zch,), jnp.float32)
    return deg_kernel(row, w, zeros)


def _sc_edge_pass(row, col, w, dis, labx, npad, ept, wtrows):
    hw = npad // 2
    wtsize = wtrows * hw
    zb = wtsize // _NS
    nwin = ept // _KB
    nown = wtrows - 1

    @functools.partial(
        pl.kernel,
        out_type=jax.ShapeDtypeStruct((_NC, 2, wtsize), jnp.float32),
        mesh=_sc_mesh(),
        scratch_types=[
            pltpu.VMEM((npad,), jnp.float32),
            pltpu.VMEM((npad,), jnp.int32),
            pltpu.VMEM((_KB,), jnp.int32),
            pltpu.VMEM((_KB,), jnp.int32),
            pltpu.VMEM((_KB,), jnp.float32),
            pltpu.VMEM((_KB,), jnp.float32),
            pltpu.VMEM((_KB,), jnp.int32),
            pltpu.VMEM_SHARED((wtsize,), jnp.float32),
        ],
        compiler_params=pltpu.CompilerParams(needs_layout_passes=False),
    )
    def edge_kernel(row_h, col_h, w_h, dis_h, labx_h, z_h, out_h,
                    dis_t, labx_t, row_v, col_v, w_v, val_v, idx_v, wt_sh):
        cid = lax.axis_index("c")
        sid = lax.axis_index("s")
        base = sid * ept
        cls0 = cid * nown
        pltpu.sync_copy(dis_h, dis_t)
        pltpu.sync_copy(labx_h, labx_t)

        for ph in range(2):
            pltpu.sync_copy(z_h, wt_sh.at[pl.ds(sid * zb, zb)])
            plsc.subcore_barrier()

            def win_body(wi, carry):
                off = base + wi * _KB
                pltpu.sync_copy(row_h.at[pl.ds(off, _KB)], row_v)
                pltpu.sync_copy(col_h.at[pl.ds(off, _KB)], col_v)
                pltpu.sync_copy(w_h.at[pl.ds(off, _KB)], w_v)

                def cbody(i, c2):
                    s = pl.ds(i * _L, _L)
                    r = row_v[s]
                    c = col_v[s]
                    wv = w_v[s]
                    dr = plsc.load_gather(dis_t, [r])
                    dc = plsc.load_gather(dis_t, [c])
                    lb = plsc.load_gather(labx_t, [r])
                    rel = lb - cls0
                    inphase = (c >= ph * hw) & (c < ph * hw + hw)
                    use = (rel >= 0) & (rel < nown) & inphase
                    lrow = jnp.where(use, rel, nown)
                    lcol = jnp.where(c >= hw, c - hw, c)
                    val_v[s] = dr * wv * dc
                    idx_v[s] = lrow * hw + lcol
                    return c2

                lax.fori_loop(0, _KB // _L, cbody, 0)
                pltpu.sync_copy(val_v, wt_sh.at[idx_v], add=True)
                return carry

            lax.fori_loop(0, nwin, win_body, 0)
            plsc.subcore_barrier()
            pltpu.sync_copy(
                wt_sh.at[pl.ds(sid * zb, zb)],
                out_h.at[cid, ph, pl.ds(sid * zb, zb)],
            )

    zeros = jnp.zeros((zb,), jnp.float32)
    return edge_kernel(row, col, w, dis, labx, zeros)


def _tc_prep(deg_a, deg_b, y3, maskf, ncls):
    def body(deg_a_ref, deg_b_ref, y3_ref, m_ref, dis_ref, labx_ref):
        deg = deg_a_ref[...] + deg_b_ref[...]
        dis = jnp.where(deg > 0, lax.rsqrt(jnp.maximum(deg, 1e-12)), 0.0)
        dis_ref[...] = dis
        lab = jnp.zeros_like(deg)
        for c in range(1, ncls):
            lab = lab + float(c) * y3_ref[c]
        labx_ref[...] = jnp.where(m_ref[...] > 0.5, lab.astype(jnp.int32), ncls)

    return pl.pallas_call(
        body,
        out_shape=(
            jax.ShapeDtypeStruct(deg_a.shape, jnp.float32),
            jax.ShapeDtypeStruct(deg_a.shape, jnp.int32),
        ),
    )(deg_a, deg_b, y3, maskf)


def _tc_final(wt_full, x_t, y_t, m_t, ncls):
    def body(wt_ref, xt_ref, yt_ref, mt_ref, h_ref):
        x = xt_ref[...]
        xm = jnp.max(x, axis=0, keepdims=True)
        e = jnp.exp(x - xm)
        probs = e / jnp.sum(e, axis=0, keepdims=True)
        m = mt_ref[...]
        yv = yt_ref[...]
        mixed = probs * (1.0 - m) + yv * m
        counts = jnp.sum(yv * m, axis=1, keepdims=True)
        wt = wt_ref[...]
        sums = lax.dot_general(
            wt, mixed, (((1,), (1,)), ((), ())),
            preferred_element_type=jnp.float32,
        )
        h = sums[:ncls, :] / counts
        hn = jnp.isnan(h)
        h = jnp.where(hn, h.T, h)
        hn = jnp.isnan(h)
        h = jnp.where(hn, 0.0, h)
        miss = (1.0 - jnp.sum(h, axis=1, keepdims=True)) / jnp.sum(
            hn.astype(jnp.float32), axis=1, keepdims=True
        )
        h = jnp.where(hn, miss, h)

        def one(i, hh):
            hh = hh / jnp.sum(hh, axis=0, keepdims=True)
            hh = hh / jnp.sum(hh, axis=1, keepdims=True)
            return hh

        def chunk(state):
            hh, it = state
            return lax.fori_loop(0, 60, one, hh), it + 60

        def cond(state):
            hh, it = state
            err = jnp.max(jnp.abs(jnp.sum(hh, axis=0) - 1.0))
            return jnp.logical_and(it < 3000, err > 5e-7)

        h, _ = lax.while_loop(cond, chunk, (h, jnp.int32(0)))
        h_ref[...] = h

    return pl.pallas_call(
        body,
        out_shape=jax.ShapeDtypeStruct((ncls, ncls), jnp.float32),
    )(wt_full, x_t, y_t, m_t)


def kernel(edge_index, edge_weight, y, inputs, sample_mask):
    n, ncls = inputs.shape
    e = edge_index.shape[1]

    npad = -(-n // 2048) * 2048
    nown = -(-(ncls + 1) // _NC)
    wtrows = nown + 1

    row = edge_index[0]
    col = edge_index[1]
    w = edge_weight

    grain = _NS * _KB
    epad = -(-e // grain) * grain
    if epad != e:
        pad = epad - e
        row = jnp.pad(row, (0, pad))
        col = jnp.pad(col, (0, pad))
        w = jnp.pad(w, (0, pad))
    epw = epad // _NW
    ept = epad // _NS

    deg2 = _sc_degree(row, w, npad, epw)

    rows128 = npad // 128
    maskf = jnp.pad(sample_mask.astype(jnp.float32), (0, npad - n))
    y_tp = jnp.pad(y, ((0, npad - n), (0, 0))).T
    y3 = y_tp.reshape(ncls, rows128, 128)
    dis, labx = _tc_prep(
        deg2[0].reshape(rows128, 128),
        deg2[1].reshape(rows128, 128),
        y3,
        maskf.reshape(rows128, 128),
        ncls,
    )

    wt2 = _sc_edge_pass(
        row, col, w, dis.reshape(npad), labx.reshape(npad), npad, ept, wtrows
    )
    hw = npad // 2
    wt_a = jnp.concatenate(
        [wt2[0, 0].reshape(wtrows, hw), wt2[0, 1].reshape(wtrows, hw)], axis=1
    )
    wt_b = jnp.concatenate(
        [wt2[1, 0].reshape(wtrows, hw), wt2[1, 1].reshape(wtrows, hw)], axis=1
    )
    wt_full = jnp.concatenate([wt_a[:nown], wt_b[: ncls + 1 - nown]], axis=0)

    x_tp = jnp.pad(inputs, ((0, npad - n), (0, 0))).T
    h = _tc_final(
        wt_full,
        x_tp,
        y_tp,
        maskf.reshape(1, npad),
        ncls,
    )
    return h
